# baseline (speedup 1.0000x reference)
import functools
import math

import jax
import jax.numpy as jnp
import numpy as np
from jax import lax
from jax.experimental import pallas as pl
from jax.experimental.pallas import tpu as pltpu
from jax.experimental.pallas import tpu_sc as plsc

_LANES = 16


def _make_pe(max_len: int, d_model: int) -> np.ndarray:
    position = np.arange(max_len, dtype=np.float32)[:, None]
    div_term = np.exp(
        np.arange(0, d_model, 2, dtype=np.float32) * (-(math.log(10000.0) / d_model))
    )
    pe = np.zeros((max_len, d_model), dtype=np.float32)
    pe[:, 0::2] = np.sin(position * div_term)
    pe[:, 1::2] = np.cos(position * div_term)
    return pe


def _pack_rows_bf16(x: jax.Array) -> jax.Array:
    r, d = x.shape
    xb = x.astype(jnp.bfloat16).reshape(r, d // 32, 2, _LANES)
    xb = xb.transpose(0, 1, 3, 2).reshape(r, d // 2, 2)
    return jax.lax.bitcast_convert_type(xb, jnp.int32)


@functools.cache
def _build_sc_kernel(N: int, D: int, V: int, C: int):
    info = plsc.get_sparse_core_info()
    NC, NS = info.num_cores, info.num_subcores
    NW = NC * NS
    assert N % NW == 0
    rows_per_w = N // NW
    CH = 16
    NSLOT = 4
    assert rows_per_w % (NSLOT * CH) == 0
    n_chunks = rows_per_w // CH

    mesh = plsc.VectorSubcoreMesh(core_axis_name="c", subcore_axis_name="s")

    @functools.partial(
        pl.kernel,
        mesh=mesh,
        compiler_params=pltpu.CompilerParams(needs_layout_passes=False),
        out_type=jax.ShapeDtypeStruct((N, D), jnp.float32),
        scratch_types=[
            pltpu.VMEM((rows_per_w,), jnp.int32),
            pltpu.VMEM((rows_per_w,), jnp.int32),
        ]
        + [pltpu.VMEM((CH, D), jnp.float32) for _ in range(NSLOT)]
        + [pltpu.VMEM((CH, D // 2), jnp.int32) for _ in range(NSLOT)]
        + [pltpu.VMEM((CH, D // 2), jnp.int32) for _ in range(NSLOT)]
        + [pltpu.SemaphoreType.DMA for _ in range(2 * NSLOT)],
    )
    def k(tok_hbm, cid_hbm, wte_hbm, combo_hbm, out_hbm, *refs):
        ti_all, ci_all = refs[0], refs[1]
        bufs_a = refs[2:2 + NSLOT]
        bufs_t = refs[2 + NSLOT:2 + 2 * NSLOT]
        bufs_b = refs[2 + 2 * NSLOT:2 + 3 * NSLOT]
        sems_g = refs[2 + 3 * NSLOT:2 + 4 * NSLOT]
        sems_s = refs[2 + 4 * NSLOT:2 + 5 * NSLOT]

        wid = lax.axis_index("s") * NC + lax.axis_index("c")
        base0 = wid * rows_per_w

        pltpu.sync_copy(tok_hbm.at[pl.ds(base0, rows_per_w)], ti_all)
        pltpu.sync_copy(cid_hbm.at[pl.ds(base0, rows_per_w)], ci_all)

        def start_gathers(i, slot):
            ti = ti_all.at[pl.ds(i * CH, CH)]
            ci = ci_all.at[pl.ds(i * CH, CH)]
            pltpu.async_copy(wte_hbm.at[ti], bufs_t[slot], sems_g[slot])
            pltpu.async_copy(combo_hbm.at[ci], bufs_b[slot], sems_g[slot])

        def wait_gathers(slot):
            pltpu.make_async_copy(
                wte_hbm.at[ti_all.at[pl.ds(0, CH)]], bufs_t[slot], sems_g[slot]).wait()
            pltpu.make_async_copy(
                combo_hbm.at[ci_all.at[pl.ds(0, CH)]], bufs_b[slot], sems_g[slot]).wait()

        def start_scatter(i, slot):
            pltpu.async_copy(
                bufs_a[slot], out_hbm.at[pl.ds(base0 + i * CH, CH)], sems_s[slot])

        def wait_scatter(slot):
            pltpu.make_async_copy(
                bufs_a[slot], out_hbm.at[pl.ds(base0, CH)], sems_s[slot]).wait()

        def add_chunk(slot):
            buf_a, buf_t, buf_b = bufs_a[slot], bufs_t[slot], bufs_b[slot]

            def add_row(r):
                for g in range(D // 32):
                    sl = pl.ds(g * _LANES, _LANES)
                    tok_sp = plsc.bitcast(buf_t[r, sl], jnp.bfloat16)
                    cmb_sp = plsc.bitcast(buf_b[r, sl], jnp.bfloat16)
                    lo_t, hi_t = plsc.unpack(
                        tok_sp, format=plsc.PackFormat.INTERLEAVED)
                    lo_c, hi_c = plsc.unpack(
                        cmb_sp, format=plsc.PackFormat.INTERLEAVED)
                    buf_a[r, pl.ds(2 * g * _LANES, _LANES)] = lo_t + lo_c
                    buf_a[r, pl.ds((2 * g + 1) * _LANES, _LANES)] = hi_t + hi_c

            plsc.parallel_loop(0, CH, 1, unroll=2)(add_row)

        start_gathers(0, 0)
        start_gathers(1, 1)

        def pipe_body(j, carry):
            for t in range(NSLOT):
                i = NSLOT * j + t
                wait_gathers(t)
                add_chunk(t)
                start_scatter(i, t)
                nslot = (t + 2) % NSLOT

                if t < 2:
                    @pl.when(j >= 1)
                    def _():
                        wait_scatter(nslot)
                    start_gathers(i + 2, nslot)
                else:
                    wait_scatter(nslot)

                    @pl.when(j < n_chunks // NSLOT - 1)
                    def _():
                        start_gathers(i + 2, nslot)
            return carry

        lax.fori_loop(0, n_chunks // NSLOT, pipe_body, 0, unroll=False)
        wait_scatter(2)
        wait_scatter(3)

    return k


def kernel(sequence, seqment_label, wte, wse):
    B, S = sequence.shape
    V, D = wte.shape
    N = B * S
    C = 3 * S

    pe = jnp.asarray(_make_pe(S, D))
    combo = (pe[:, None, :] + wse[None, :, :]).reshape(C, D)
    combo_bf = _pack_rows_bf16(combo)
    wte_bf = _pack_rows_bf16(wte)

    tok_idx = sequence.reshape(N).astype(jnp.int32)
    cid = (
        3 * jnp.arange(S, dtype=jnp.int32)[None, :]
        + seqment_label.astype(jnp.int32)
    ).reshape(N)

    k = _build_sc_kernel(N, D, V, C)
    out = k(tok_idx, cid, wte_bf, combo_bf)
    return out.reshape(B, S, D)


# speedup vs baseline: 1.5890x; 1.5890x over previous
import functools
import math

import jax
import jax.numpy as jnp
import numpy as np
from jax import lax
from jax.experimental import pallas as pl
from jax.experimental.pallas import tpu as pltpu
from jax.experimental.pallas import tpu_sc as plsc

_LANES = 16


def _make_pe(max_len: int, d_model: int) -> np.ndarray:
    position = np.arange(max_len, dtype=np.float32)[:, None]
    div_term = np.exp(
        np.arange(0, d_model, 2, dtype=np.float32) * (-(math.log(10000.0) / d_model))
    )
    pe = np.zeros((max_len, d_model), dtype=np.float32)
    pe[:, 0::2] = np.sin(position * div_term)
    pe[:, 1::2] = np.cos(position * div_term)
    return pe


def _interleave_perm(d: int) -> np.ndarray:
    perm = np.empty((d,), dtype=np.int32)
    for b in range(d // 32):
        for l in range(_LANES):
            perm[32 * b + 2 * l] = 32 * b + l
            perm[32 * b + 2 * l + 1] = 32 * b + 16 + l
    return perm


@functools.cache
def _build_sc_kernel(N: int, D: int, V: int, C: int):
    info = plsc.get_sparse_core_info()
    NC, NS = info.num_cores, info.num_subcores
    NW = NC * NS
    assert N % NW == 0
    rows_per_w = N // NW
    CH = 32
    NSLOT = 4
    assert rows_per_w % (NSLOT * CH) == 0
    n_chunks = rows_per_w // CH

    mesh = plsc.VectorSubcoreMesh(core_axis_name="c", subcore_axis_name="s")

    @functools.partial(
        pl.kernel,
        mesh=mesh,
        compiler_params=pltpu.CompilerParams(needs_layout_passes=False),
        out_type=jax.ShapeDtypeStruct((N, D), jnp.float32),
        scratch_types=[
            pltpu.VMEM((rows_per_w,), jnp.int32),
            pltpu.VMEM((rows_per_w,), jnp.int32),
        ]
        + [pltpu.VMEM((CH, D), jnp.float32) for _ in range(NSLOT)]
        + [pltpu.VMEM((CH, D // 2), jnp.int32) for _ in range(NSLOT)]
        + [pltpu.SemaphoreType.DMA for _ in range(2 * NSLOT)],
    )
    def k(tok_hbm, cid_hbm, wte_hbm, combo_hbm, out_hbm, *refs):
        ti_all, ci_all = refs[0], refs[1]
        bufs_a = refs[2:2 + NSLOT]
        bufs_b = refs[2 + NSLOT:2 + 2 * NSLOT]
        sems_g = refs[2 + 2 * NSLOT:2 + 3 * NSLOT]
        sems_s = refs[2 + 3 * NSLOT:2 + 4 * NSLOT]

        wid = lax.axis_index("s") * NC + lax.axis_index("c")
        base0 = wid * rows_per_w

        pltpu.sync_copy(tok_hbm.at[pl.ds(base0, rows_per_w)], ti_all)
        pltpu.sync_copy(cid_hbm.at[pl.ds(base0, rows_per_w)], ci_all)

        def start_gathers(i, slot):
            ti = ti_all.at[pl.ds(i * CH, CH)]
            ci = ci_all.at[pl.ds(i * CH, CH)]
            pltpu.async_copy(wte_hbm.at[ti], bufs_a[slot], sems_g[slot])
            pltpu.async_copy(combo_hbm.at[ci], bufs_b[slot], sems_g[slot])

        def wait_gathers(slot):
            pltpu.make_async_copy(
                wte_hbm.at[ti_all.at[pl.ds(0, CH)]], bufs_a[slot], sems_g[slot]).wait()
            pltpu.make_async_copy(
                combo_hbm.at[ci_all.at[pl.ds(0, CH)]], bufs_b[slot], sems_g[slot]).wait()

        def start_scatter(i, slot):
            pltpu.async_copy(
                bufs_a[slot], out_hbm.at[pl.ds(base0 + i * CH, CH)], sems_s[slot])

        def wait_scatter(slot):
            pltpu.make_async_copy(
                bufs_a[slot], out_hbm.at[pl.ds(base0, CH)], sems_s[slot]).wait()

        def add_chunk(slot):
            buf_a, buf_b = bufs_a[slot], bufs_b[slot]

            def add_row(r):
                for g in range(D // 32):
                    sp = plsc.bitcast(
                        buf_b[r, pl.ds(g * _LANES, _LANES)], jnp.bfloat16)
                    lo, hi = plsc.unpack(sp, format=plsc.PackFormat.INTERLEAVED)
                    sl0 = pl.ds(2 * g * _LANES, _LANES)
                    sl1 = pl.ds((2 * g + 1) * _LANES, _LANES)
                    buf_a[r, sl0] = buf_a[r, sl0] + lo
                    buf_a[r, sl1] = buf_a[r, sl1] + hi

            plsc.parallel_loop(0, CH, 1, unroll=2)(add_row)

        start_gathers(0, 0)
        start_gathers(1, 1)

        def pipe_body(j, carry):
            for t in range(NSLOT):
                i = NSLOT * j + t
                wait_gathers(t)
                add_chunk(t)
                start_scatter(i, t)
                nslot = (t + 2) % NSLOT

                if t < 2:
                    @pl.when(j >= 1)
                    def _():
                        wait_scatter(nslot)
                    start_gathers(i + 2, nslot)
                else:
                    wait_scatter(nslot)

                    @pl.when(j < n_chunks // NSLOT - 1)
                    def _():
                        start_gathers(i + 2, nslot)
            return carry

        lax.fori_loop(0, n_chunks // NSLOT, pipe_body, 0, unroll=False)
        wait_scatter(2)
        wait_scatter(3)

    return k


def kernel(sequence, seqment_label, wte, wse):
    B, S = sequence.shape
    V, D = wte.shape
    N = B * S
    C = 3 * S

    pe = jnp.asarray(_make_pe(S, D))
    combo = (pe[:, None, :] + wse[None, :, :]).reshape(C, D)
    perm = _interleave_perm(D)
    combo_bf = jax.lax.bitcast_convert_type(
        combo.astype(jnp.bfloat16)[:, perm].reshape(C, D // 2, 2), jnp.int32
    )

    tok_idx = sequence.reshape(N).astype(jnp.int32)
    cid = (
        3 * jnp.arange(S, dtype=jnp.int32)[None, :]
        + seqment_label.astype(jnp.int32)
    ).reshape(N)

    k = _build_sc_kernel(N, D, V, C)
    out = k(tok_idx, cid, wte, combo_bf)
    return out.reshape(B, S, D)
